# baseline (speedup 1.0000x reference)
import functools

import jax
import jax.numpy as jnp
from jax import lax
from jax.experimental import pallas as pl
from jax.experimental.pallas import tpu as pltpu
from jax.experimental.pallas import tpu_sc as plsc

_F = 26
_V = 100000
_E = 32
_B = 16384
_BS = _B // 16
_CHUNKS = (7, 7, 6, 6)

_mesh = plsc.VectorSubcoreMesh(core_axis_name="c", subcore_axis_name="s")


@functools.cache
def _chunk_kernel(base_f: int, nf: int):
    @functools.partial(
        pl.kernel,
        mesh=_mesh,
        compiler_params=pltpu.CompilerParams(use_tc_tiling_on_sc=False),
        out_type=(),
        scratch_types=[
            pltpu.VMEM_SHARED((2, 8, _V), jnp.float32),
            pltpu.VMEM((_BS,), jnp.int32),
            pltpu.VMEM((2, 8, _BS), jnp.float32),
            pltpu.SemaphoreType.DMA,
            pltpu.SemaphoreType.DMA,
            pltpu.SemaphoreType.DMA,
            pltpu.SemaphoreType.DMA,
            pltpu.SemaphoreType.DMA,
        ],
    )
    def _emb_chunk(x_hbm, tab_hbm, out_hbm, sbuf, vidx, obuf,
                   sem_t0, sem_t1, sem_o0, sem_o1, sem_g):
        c = lax.axis_index("c")
        s = lax.axis_index("s")
        b0 = s * _BS
        sem_t = (sem_t0, sem_t1)
        sem_o = (sem_o0, sem_o1)

        def stage(g, b, buf):
            return pltpu.async_copy(
                tab_hbm.at[pl.ds((4 * g + 2 * b + c) * 8, 8), :],
                sbuf.at[buf],
                sem_t[buf],
            )

        @pl.when(s == 0)
        def _prologue():
            stage(0, 0, 0)

        def pair(g, carry):
            for b in (0, 1):
                n = 2 * g + b

                @pl.when(s == 0)
                def _wait_stage():
                    pltpu.make_async_copy(
                        tab_hbm.at[pl.ds(0, 8), :], sbuf.at[b], sem_t[b]
                    ).wait()

                plsc.subcore_barrier()

                @pl.when((s == 0) & (n + 1 < 2 * nf))
                def _stage_next():
                    if b == 0:
                        stage(g, 1, 1)
                    else:
                        stage(g + 1, 0, 0)

                if b == 0:
                    pltpu.sync_copy(
                        x_hbm.at[pl.ds((base_f + g) * _B + b0, _BS)], vidx
                    )

                @pl.when(n >= 2)
                def _wait_out():
                    pltpu.make_async_copy(
                        obuf.at[b],
                        out_hbm.at[pl.ds(0, 8), pl.ds(0, _BS)],
                        sem_o[b],
                    ).wait()

                copies = [
                    pltpu.async_copy(
                        sbuf.at[b].at[e].at[vidx], obuf.at[b, e], sem_g
                    )
                    for e in range(8)
                ]
                for cp in copies:
                    cp.wait()
                r0 = base_f * _E + (4 * g + 2 * b + c) * 8
                pltpu.async_copy(
                    obuf.at[b],
                    out_hbm.at[pl.ds(r0, 8), pl.ds(b0, _BS)],
                    sem_o[b],
                )
                plsc.subcore_barrier()
            return carry

        lax.fori_loop(0, nf, pair, 0)
        pltpu.make_async_copy(
            obuf.at[0], out_hbm.at[pl.ds(0, 8), pl.ds(0, _BS)], sem_o[0]
        ).wait()
        pltpu.make_async_copy(
            obuf.at[1], out_hbm.at[pl.ds(0, 8), pl.ds(0, _BS)], sem_o[1]
        ).wait()

    return _emb_chunk


def kernel(x, tables):
    xt = jnp.swapaxes(x, 0, 1).reshape(_F * _B).astype(jnp.int32)
    out_ref = jax.new_ref(jnp.empty((_F * _E, _B), jnp.float32))
    base = 0
    for nf in _CHUNKS:
        tab_j = jnp.swapaxes(tables[base:base + nf], 1, 2).reshape(nf * _E, _V)
        _chunk_kernel(base, nf)(xt, tab_j, out_ref)
        base += nf
    out2 = out_ref[...]
    return out2.reshape(_F, _E, _B).transpose(2, 0, 1)


# speedup vs baseline: 1.1488x; 1.1488x over previous
import functools

import jax
import jax.numpy as jnp
from jax import lax
from jax.experimental import pallas as pl
from jax.experimental.pallas import tpu as pltpu
from jax.experimental.pallas import tpu_sc as plsc

_F = 26
_V = 100000
_E = 32
_B = 16384
_FC = 13
_NB = _FC * 4
_BS = _B // 16

_mesh = plsc.VectorSubcoreMesh(core_axis_name="c", subcore_axis_name="s")


@functools.partial(
    pl.kernel,
    mesh=_mesh,
    compiler_params=pltpu.CompilerParams(use_tc_tiling_on_sc=False),
    out_type=jax.ShapeDtypeStruct((_F, _E, _B), jnp.float32),
    scratch_types=[
        pltpu.VMEM_SHARED((2, 8, _V), jnp.float32),
        pltpu.VMEM((_BS,), jnp.int32),
        pltpu.VMEM((2, 8, _BS), jnp.float32),
        pltpu.SemaphoreType.DMA,
        pltpu.SemaphoreType.DMA,
        pltpu.SemaphoreType.DMA,
        pltpu.SemaphoreType.DMA,
        pltpu.SemaphoreType.DMA,
    ],
)
def _emb_kernel(x_hbm, tab_hbm, out_hbm, sbuf, vidx, obuf,
                sem_t0, sem_t1, sem_o0, sem_o1, sem_g):
    c = lax.axis_index("c")
    s = lax.axis_index("s")
    b0 = s * _BS
    sem_t = (sem_t0, sem_t1)
    sem_o = (sem_o0, sem_o1)

    def stage(n, buf):
        return pltpu.async_copy(
            tab_hbm.at[pl.ds((_NB * c + n) * 8, 8), :], sbuf.at[buf], sem_t[buf]
        )

    @pl.when(s == 0)
    def _prologue():
        stage(0, 0)

    def pair(g, carry):
        for b in (0, 1):
            n = 2 * g + b

            @pl.when(s == 0)
            def _wait_stage():
                pltpu.make_async_copy(
                    tab_hbm.at[pl.ds(0, 8), :], sbuf.at[b], sem_t[b]
                ).wait()

            plsc.subcore_barrier()

            @pl.when((s == 0) & (n + 1 < _NB))
            def _stage_next():
                stage(n + 1, 1 - b)

            if b == 0:
                @pl.when(g % 2 == 0)
                def _load_idx():
                    f = _FC * c + n // 4
                    pltpu.sync_copy(x_hbm.at[pl.ds(f * _B + b0, _BS)], vidx)

            @pl.when(n >= 2)
            def _wait_out():
                pltpu.make_async_copy(
                    obuf.at[b],
                    out_hbm.at[0, pl.ds(0, 8), pl.ds(0, _BS)],
                    sem_o[b],
                ).wait()

            copies = [
                pltpu.async_copy(
                    sbuf.at[b].at[e].at[vidx], obuf.at[b, e], sem_g
                )
                for e in range(8)
            ]
            for cp in copies:
                cp.wait()
            m = _NB * c + n
            pltpu.async_copy(
                obuf.at[b],
                out_hbm.at[m // 4, pl.ds((m % 4) * 8, 8), pl.ds(b0, _BS)],
                sem_o[b],
            )
            plsc.subcore_barrier()
        return carry

    lax.fori_loop(0, _NB // 2, pair, 0)
    pltpu.make_async_copy(
        obuf.at[0], out_hbm.at[0, pl.ds(0, 8), pl.ds(0, _BS)], sem_o[0]
    ).wait()
    pltpu.make_async_copy(
        obuf.at[1], out_hbm.at[0, pl.ds(0, 8), pl.ds(0, _BS)], sem_o[1]
    ).wait()


def kernel(x, tables):
    xt = jnp.swapaxes(x, 0, 1).reshape(_F * _B).astype(jnp.int32)
    tab2 = jnp.swapaxes(tables, 1, 2).reshape(_F * _E, _V)
    out3 = _emb_kernel(xt, tab2)
    return out3.transpose(2, 0, 1)
